# baseline (speedup 1.0000x reference)
import functools

import jax
import jax.numpy as jnp
from jax import lax
from jax.experimental import pallas as pl
from jax.experimental.pallas import tpu as pltpu
from jax.experimental.pallas import tpu_sc as plsc

NUM_CORES = 2
NUM_SUBCORES = 16
NW = NUM_CORES * NUM_SUBCORES
BATCH = 16384
BPW = BATCH // NW
NCHUNK = BPW // 128

ALPHA = 4.0
BETA = 400.0

_LOG1P_C = (
    2.2159764915272586e-07,
    0.9999702432977375,
    -0.49933394898195,
    0.3275117137018214,
    -0.22396689942949047,
    0.13198966239919324,
    -0.05326747773338258,
    0.010243828631143087,
)


def _log1p01(t):
    acc = jnp.full((16,), _LOG1P_C[-1], jnp.float32)
    for c in _LOG1P_C[-2::-1]:
        acc = acc * t + jnp.float32(c)
    return acc


def _sc_body(inst_hbm, pred_hbm, emb_hbm, qi_hbm, qp_hbm,
             preds_out, q_out, reg_out,
             idx_i, idx_p, blk_e, blk_q, blk_p, emb_v, qi_v, qp_v,
             preds_v, q_v, acc_v, shloc_v, out16_v, shared_v, sem):
    cid = lax.axis_index("c")
    sid = lax.axis_index("s")
    wid = cid * NUM_SUBCORES + sid
    base = wid * BPW
    lanes = lax.iota(jnp.int32, 16)

    cps = []
    for j in range(NCHUNK):
        cps.append(pltpu.async_copy(
            inst_hbm.at[pl.ds(base + 128 * j, 128)], idx_i.at[j], sem))
        cps.append(pltpu.async_copy(
            pred_hbm.at[pl.ds(base + 128 * j, 128)], idx_p.at[j], sem))
    for cp in cps:
        cp.wait()

    def blk_body(c, carry):
        row = jnp.full((16,), lax.shift_right_logical(c, 3), jnp.int32)
        col = lax.bitwise_and(c, 7) * 16 + lanes
        vi = plsc.load_gather(idx_i, [row, col])
        vp = plsc.load_gather(idx_p, [row, col])
        plsc.store_scatter(blk_e, [row, col], lax.shift_right_logical(vi, 3))
        plsc.store_scatter(blk_q, [row, col], lax.shift_right_logical(vi, 2))
        plsc.store_scatter(blk_p, [row, col], lax.shift_right_logical(vp, 2))
        return carry

    lax.fori_loop(0, NCHUNK * 8, blk_body, 0)

    cps = []
    for j in range(NCHUNK):
        cps.append(pltpu.async_copy(emb_hbm.at[blk_e.at[j]], emb_v.at[j], sem))
        cps.append(pltpu.async_copy(qi_hbm.at[blk_q.at[j]], qi_v.at[j], sem))
        cps.append(pltpu.async_copy(qp_hbm.at[blk_p.at[j]], qp_v.at[j], sem))
    for cp in cps:
        cp.wait()

    def pred_body(c, carry):
        flat = c * 16 + lanes
        i = lax.shift_right_logical(flat, 1)
        cc = lax.bitwise_and(flat, 1)
        ji = lax.shift_right_logical(i, 7)
        ci = lax.bitwise_and(i, 127)
        rv = plsc.load_gather(idx_i, [ji, ci])
        word = lax.bitwise_and(rv, 7) * 2 + cc
        x = plsc.load_gather(emb_v, [ji, ci, word])
        t = jnp.exp(-jnp.abs(x))
        preds_v[pl.ds(pl.multiple_of(c * 16, 16), 16)] = (
            jnp.minimum(x, 0.0) - _log1p01(t))
        return carry

    lax.fori_loop(0, 64, pred_body, 0)

    def q_body(c, carry):
        ai, ap = carry
        flat = c * 16 + lanes
        i = lax.shift_right_logical(flat, 2)
        cc = lax.bitwise_and(flat, 3)
        ji = lax.shift_right_logical(i, 7)
        ci = lax.bitwise_and(i, 127)
        rvi = plsc.load_gather(idx_i, [ji, ci])
        rvp = plsc.load_gather(idx_p, [ji, ci])
        wi = lax.bitwise_and(rvi, 3) * 4 + cc
        wp = lax.bitwise_and(rvp, 3) * 4 + cc
        vi = plsc.load_gather(qi_v, [ji, ci, wi])
        vp = plsc.load_gather(qp_v, [ji, ci, wp])
        pi = plsc.load_gather(qi_v, [ji, ci, lax.bitwise_xor(wi, 1)])
        pp = plsc.load_gather(qp_v, [ji, ci, lax.bitwise_xor(wp, 1)])
        q = vi + vp
        qq = pi + pp
        m = jnp.maximum(q, qq)
        t = jnp.exp(-jnp.abs(q - qq))
        q_v[pl.ds(pl.multiple_of(c * 16, 16), 16)] = q - m - _log1p01(t)
        return ai + vi * vi, ap + vp * vp

    zero = jnp.zeros((16,), jnp.float32)
    acc_i, acc_p = lax.fori_loop(0, 128, q_body, (zero, zero))

    cp_p = pltpu.async_copy(preds_v, preds_out.at[pl.ds(wid * 1024, 1024)], sem)
    cp_q = pltpu.async_copy(q_v, q_out.at[pl.ds(wid * 2048, 2048)], sem)

    acc_v[pl.ds(0, 16)] = acc_i
    acc_v[pl.ds(16, 16)] = acc_p
    pltpu.sync_copy(acc_v, shared_v.at[sid])
    plsc.subcore_barrier()

    @pl.when(sid == 0)
    def _():
        pltpu.sync_copy(shared_v, shloc_v)
        zi = jnp.zeros((16,), jnp.float32)
        zp = jnp.zeros((16,), jnp.float32)
        for i in range(NUM_SUBCORES):
            row = jnp.full((16,), i, jnp.int32)
            zi = zi + plsc.load_gather(shloc_v, [row, lanes])
            zp = zp + plsc.load_gather(shloc_v, [row, lanes + 16])
        ri = jnp.sum(zi) * jnp.float32(BETA / 2.0)
        rp = jnp.sum(zp) * jnp.float32(ALPHA / 2.0)
        out16_v[pl.ds(0, 16)] = jnp.where(
            lanes == 0, ri, jnp.where(lanes == 1, rp, jnp.float32(0.0)))
        pltpu.sync_copy(out16_v, reg_out.at[cid])

    cp_p.wait()
    cp_q.wait()


_sc_call = functools.partial(
    pl.kernel,
    out_type=[
        jax.ShapeDtypeStruct((BATCH * 2,), jnp.float32),
        jax.ShapeDtypeStruct((BATCH * 4,), jnp.float32),
        jax.ShapeDtypeStruct((NUM_CORES, 16), jnp.float32),
    ],
    compiler_params=pltpu.CompilerParams(
        needs_layout_passes=False, use_tc_tiling_on_sc=False),
    mesh=plsc.VectorSubcoreMesh(
        core_axis_name="c", subcore_axis_name="s",
        num_cores=NUM_CORES, num_subcores=NUM_SUBCORES),
    scratch_types=[
        pltpu.VMEM((NCHUNK, 128), jnp.int32),
        pltpu.VMEM((NCHUNK, 128), jnp.int32),
        pltpu.VMEM((NCHUNK, 128), jnp.int32),
        pltpu.VMEM((NCHUNK, 128), jnp.int32),
        pltpu.VMEM((NCHUNK, 128), jnp.int32),
        pltpu.VMEM((NCHUNK, 128, 16), jnp.float32),
        pltpu.VMEM((NCHUNK, 128, 16), jnp.float32),
        pltpu.VMEM((NCHUNK, 128, 16), jnp.float32),
        pltpu.VMEM((BPW * 2,), jnp.float32),
        pltpu.VMEM((BPW * 4,), jnp.float32),
        pltpu.VMEM((32,), jnp.float32),
        pltpu.VMEM((NUM_SUBCORES, 32), jnp.float32),
        pltpu.VMEM((16,), jnp.float32),
        pltpu.VMEM_SHARED((NUM_SUBCORES, 32), jnp.float32),
        pltpu.SemaphoreType.DMA,
    ],
)(_sc_body)


def kernel(instances, predictors, labels, instance_emb, q_i_table, q_p_table):
    del labels
    emb_r = instance_emb.reshape(-1, 16)
    qi_r = q_i_table.reshape(-1, 16)
    qp_r = q_p_table.reshape(-1, 16)
    preds_flat, q_flat, regs = _sc_call(
        instances, predictors, emb_r, qi_r, qp_r)
    predictions = preds_flat.reshape(BATCH, 2)
    q_params = q_flat.reshape(BATCH, 2, 2)
    reg_i = regs[0, 0] + regs[1, 0]
    reg_p = regs[0, 1] + regs[1, 1]
    return predictions, q_params, reg_i, reg_p


# speedup vs baseline: 1.0037x; 1.0037x over previous
import functools

import jax
import jax.numpy as jnp
from jax import lax
from jax.experimental import pallas as pl
from jax.experimental.pallas import tpu as pltpu
from jax.experimental.pallas import tpu_sc as plsc

NUM_CORES = 2
NUM_SUBCORES = 16
NW = NUM_CORES * NUM_SUBCORES
BATCH = 16384
BPW = BATCH // NW
NCHUNK = BPW // 128
NBUF = 3

ALPHA = 4.0
BETA = 400.0

_LOG1P_C = (
    2.2159764915272586e-07,
    0.9999702432977375,
    -0.49933394898195,
    0.3275117137018214,
    -0.22396689942949047,
    0.13198966239919324,
    -0.05326747773338258,
    0.010243828631143087,
)


def _log1p01(t):
    acc = jnp.full((16,), _LOG1P_C[-1], jnp.float32)
    for c in _LOG1P_C[-2::-1]:
        acc = acc * t + jnp.float32(c)
    return acc


def _sc_body(inst_hbm, pred_hbm, emb_hbm, qi_hbm, qp_hbm,
             preds_out, q_out, reg_out,
             idx_i, idx_p, blk_e, blk_q, emb_v, qi_v, qp_loc,
             preds_v, q_v, out16_v,
             sem_qp, sem_idx, sem_g, sem_out):
    cid = lax.axis_index("c")
    sid = lax.axis_index("s")
    wid = cid * NUM_SUBCORES + sid
    base = wid * BPW
    lanes = lax.iota(jnp.int32, 16)

    cp_qp = pltpu.async_copy(qp_hbm, qp_loc.at[pl.ds(0, 4000)], sem_qp)

    cps = []
    for j in range(NCHUNK):
        cps.append(pltpu.async_copy(
            inst_hbm.at[pl.ds(base + 128 * j, 128)], idx_i.at[j], sem_idx))
        cps.append(pltpu.async_copy(
            pred_hbm.at[pl.ds(base + 128 * j, 128)], idx_p.at[j], sem_idx))
    for cp in cps:
        cp.wait()

    def blk_body(c, carry):
        row = jnp.full((16,), lax.shift_right_logical(c, 3), jnp.int32)
        col = lax.bitwise_and(c, 7) * 16 + lanes
        vi = plsc.load_gather(idx_i, [row, col])
        plsc.store_scatter(blk_e, [row, col], lax.shift_right_logical(vi, 6))
        plsc.store_scatter(blk_q, [row, col], lax.shift_right_logical(vi, 5))
        return carry

    lax.fori_loop(0, NCHUNK * 8, blk_body, 0)

    def issue(j):
        b = j % NBUF
        return (pltpu.async_copy(emb_hbm.at[blk_e.at[j]], emb_v.at[b],
                                 sem_g.at[b]),
                pltpu.async_copy(qi_hbm.at[blk_q.at[j]], qi_v.at[b],
                                 sem_g.at[b]))

    inflight = [issue(j) for j in range(NBUF)]
    cp_qp.wait()

    def pred_body_for(j, b):
        def pred_body(c, carry):
            flat = c * 16 + lanes
            i = lax.shift_right_logical(flat, 1)
            cc = lax.bitwise_and(flat, 1)
            jv = jnp.full((16,), j, jnp.int32)
            bv = jnp.full((16,), b, jnp.int32)
            rv = plsc.load_gather(idx_i, [jv, i])
            word = lax.bitwise_and(rv, 63) * 2 + cc
            x = plsc.load_gather(emb_v, [bv, i, word])
            t = jnp.exp(-jnp.abs(x))
            preds_v[pl.ds(pl.multiple_of(j * 256 + c * 16, 16), 16)] = (
                jnp.minimum(x, 0.0) - _log1p01(t))
            return carry
        return pred_body

    def q_body_for(j, b):
        def q_body(c, carry):
            ai, ap = carry
            flat = c * 16 + lanes
            i = lax.shift_right_logical(flat, 2)
            cc = lax.bitwise_and(flat, 3)
            jv = jnp.full((16,), j, jnp.int32)
            bv = jnp.full((16,), b, jnp.int32)
            rvi = plsc.load_gather(idx_i, [jv, i])
            rvp = plsc.load_gather(idx_p, [jv, i])
            wi = lax.bitwise_and(rvi, 31) * 4 + cc
            wp = rvp * 4 + cc
            vi = plsc.load_gather(qi_v, [bv, i, wi])
            vp = plsc.load_gather(qp_loc, [wp])
            pi = plsc.load_gather(qi_v, [bv, i, lax.bitwise_xor(wi, 1)])
            pp = plsc.load_gather(qp_loc, [lax.bitwise_xor(wp, 1)])
            q = vi + vp
            qq = pi + pp
            m = jnp.maximum(q, qq)
            t = jnp.exp(-jnp.abs(q - qq))
            q_v[pl.ds(pl.multiple_of(j * 512 + c * 16, 16), 16)] = (
                q - m - _log1p01(t))
            return ai + vi * vi, ap + vp * vp
        return q_body

    zero = jnp.zeros((16,), jnp.float32)
    acc_i, acc_p = zero, zero
    for j in range(NCHUNK):
        for cp in inflight[j]:
            cp.wait()
        lax.fori_loop(0, 16, pred_body_for(j, j % NBUF), 0)
        acc_i, acc_p = lax.fori_loop(
            0, 32, q_body_for(j, j % NBUF), (acc_i, acc_p))
        if j + NBUF < NCHUNK:
            inflight.append(issue(j + NBUF))

    cp_p = pltpu.async_copy(preds_v, preds_out.at[pl.ds(wid * 1024, 1024)],
                            sem_out)
    cp_q = pltpu.async_copy(q_v, q_out.at[pl.ds(wid * 2048, 2048)], sem_out)

    ri = jnp.sum(acc_i) * jnp.float32(BETA / 2.0)
    rp = jnp.sum(acc_p) * jnp.float32(ALPHA / 2.0)
    out16_v[pl.ds(0, 16)] = jnp.where(
        lanes == 0, ri, jnp.where(lanes == 1, rp, jnp.float32(0.0)))
    pltpu.sync_copy(out16_v, reg_out.at[pl.ds(wid * 16, 16)])

    cp_p.wait()
    cp_q.wait()


_sc_call = functools.partial(
    pl.kernel,
    out_type=[
        jax.ShapeDtypeStruct((BATCH * 2,), jnp.float32),
        jax.ShapeDtypeStruct((BATCH * 4,), jnp.float32),
        jax.ShapeDtypeStruct((NW * 16,), jnp.float32),
    ],
    compiler_params=pltpu.CompilerParams(
        needs_layout_passes=False, use_tc_tiling_on_sc=True),
    mesh=plsc.VectorSubcoreMesh(
        core_axis_name="c", subcore_axis_name="s",
        num_cores=NUM_CORES, num_subcores=NUM_SUBCORES),
    scratch_types=[
        pltpu.VMEM((NCHUNK, 128), jnp.int32),
        pltpu.VMEM((NCHUNK, 128), jnp.int32),
        pltpu.VMEM((NCHUNK, 128), jnp.int32),
        pltpu.VMEM((NCHUNK, 128), jnp.int32),
        pltpu.VMEM((NBUF, 128, 128), jnp.float32),
        pltpu.VMEM((NBUF, 128, 128), jnp.float32),
        pltpu.VMEM((4096,), jnp.float32),
        pltpu.VMEM((BPW * 2,), jnp.float32),
        pltpu.VMEM((BPW * 4,), jnp.float32),
        pltpu.VMEM((16,), jnp.float32),
        pltpu.SemaphoreType.DMA,
        pltpu.SemaphoreType.DMA,
        pltpu.SemaphoreType.DMA((NBUF,)),
        pltpu.SemaphoreType.DMA,
    ],
)(_sc_body)


def kernel(instances, predictors, labels, instance_emb, q_i_table, q_p_table):
    del labels
    emb_r = instance_emb.reshape(-1, 128)
    qi_r = q_i_table.reshape(-1, 128)
    qp_r = q_p_table.reshape(-1)
    preds_flat, q_flat, regs = _sc_call(
        instances, predictors, emb_r, qi_r, qp_r)
    predictions = preds_flat.reshape(BATCH, 2)
    q_params = q_flat.reshape(BATCH, 2, 2)
    parts = regs.reshape(NW, 16)
    reg_i = parts[:, 0].sum()
    reg_p = parts[:, 1].sum()
    return predictions, q_params, reg_i, reg_p


# speedup vs baseline: 12.3675x; 12.3213x over previous
import functools

import jax
import jax.numpy as jnp
from jax import lax
from jax.experimental import pallas as pl
from jax.experimental.pallas import tpu as pltpu
from jax.experimental.pallas import tpu_sc as plsc

NUM_CORES = 2
NUM_SUBCORES = 16
NW = NUM_CORES * NUM_SUBCORES
BATCH = 16384
BPW = BATCH // NW
NCHUNK = BPW // 128
NBUF = 3

NROWS = 1000000
HEAD = (NROWS // 128) * 128
NTAIL = NROWS - HEAD

ALPHA = 4.0
BETA = 400.0

_LOG1P_C = (
    2.2159764915272586e-07,
    0.9999702432977375,
    -0.49933394898195,
    0.3275117137018214,
    -0.22396689942949047,
    0.13198966239919324,
    -0.05326747773338258,
    0.010243828631143087,
)


def _log1p01(t):
    acc = jnp.full((16,), _LOG1P_C[-1], jnp.float32)
    for c in _LOG1P_C[-2::-1]:
        acc = acc * t + jnp.float32(c)
    return acc


def _sc_body(inst_hbm, pred_hbm, we_hbm, wq_hbm, etail_hbm, qtail_hbm, qp_hbm,
             preds_out, q_out, reg_out,
             idx_i, idx_p, blk, gbuf, etail_v, qtail_v, qp_v,
             preds_v, q_v, out16_v,
             sem_in, sem_g, sem_out):
    cid = lax.axis_index("c")
    sid = lax.axis_index("s")
    wid = cid * NUM_SUBCORES + sid
    base = wid * BPW
    lanes = lax.iota(jnp.int32, 16)

    cps = [pltpu.async_copy(etail_hbm, etail_v, sem_in),
           pltpu.async_copy(qtail_hbm, qtail_v, sem_in),
           pltpu.async_copy(qp_hbm, qp_v, sem_in)]
    for j in range(NCHUNK):
        cps.append(pltpu.async_copy(
            inst_hbm.at[pl.ds(base + 128 * j, 128)], idx_i.at[j], sem_in))
        cps.append(pltpu.async_copy(
            pred_hbm.at[pl.ds(base + 128 * j, 128)], idx_p.at[j], sem_in))
    for cp in cps:
        cp.wait()

    def blk_body(k, carry):
        row = jnp.full((16,), lax.shift_right_logical(k, 3), jnp.int32)
        col = lax.bitwise_and(k, 7) * 16 + lanes
        v = plsc.load_gather(idx_i, [row, col])
        t = lax.shift_right_logical(v, 7)
        lo = lax.bitwise_and(lax.shift_right_logical(v, 4), 7)
        inhead = v < HEAD
        e_base = t * 16 + lo
        q_base = t * 32 + lo
        zero = jnp.zeros((16,), jnp.int32)
        for m in range(2):
            plsc.store_scatter(
                blk, [row, jnp.full((16,), m, jnp.int32), col],
                jnp.where(inhead, e_base + 8 * m, zero))
        for m in range(4):
            plsc.store_scatter(
                blk, [row, jnp.full((16,), 2 + m, jnp.int32), col],
                jnp.where(inhead, q_base + 8 * m, zero))
        return carry

    lax.fori_loop(0, NCHUNK * 8, blk_body, 0)

    def issue(j):
        b = j % NBUF
        out = []
        for m in range(2):
            out.append(pltpu.async_copy(
                we_hbm.at[blk.at[j, m]], gbuf.at[b, m], sem_g.at[b]))
        for m in range(4):
            out.append(pltpu.async_copy(
                wq_hbm.at[blk.at[j, 2 + m]], gbuf.at[b, 2 + m], sem_g.at[b]))
        return out

    inflight = [issue(j) for j in range(NBUF)]

    def pred_body_for(j, b):
        def pred_body(c, carry):
            flat = c * 16 + lanes
            i = lax.shift_right_logical(flat, 1)
            cc = lax.bitwise_and(flat, 1)
            jv = jnp.full((16,), j, jnp.int32)
            bv = jnp.full((16,), b, jnp.int32)
            r = plsc.load_gather(idx_i, [jv, i])
            x_main = plsc.load_gather(
                gbuf, [bv, cc, i, lax.bitwise_and(r, 15)])
            tmask = r >= HEAD
            tw = jnp.maximum(r - HEAD, 0) * 2 + cc
            x_tail = plsc.load_gather(
                etail_v, [lax.shift_right_logical(tw, 4),
                          lax.bitwise_and(tw, 15)])
            x = jnp.where(tmask, x_tail, x_main)
            t = jnp.exp(-jnp.abs(x))
            preds_v[pl.ds(pl.multiple_of(j * 256 + c * 16, 16), 16)] = (
                jnp.minimum(x, 0.0) - _log1p01(t))
            return carry
        return pred_body

    def q_body_for(j, b):
        def q_body(c, carry):
            ai, ap = carry
            flat = c * 16 + lanes
            i = lax.shift_right_logical(flat, 2)
            cc = lax.bitwise_and(flat, 3)
            ccx = lax.bitwise_xor(cc, 1)
            jv = jnp.full((16,), j, jnp.int32)
            bv = jnp.full((16,), b, jnp.int32)
            r = plsc.load_gather(idx_i, [jv, i])
            rp = plsc.load_gather(idx_p, [jv, i])
            rw = lax.bitwise_and(r, 15)
            vi_m = plsc.load_gather(gbuf, [bv, 2 + cc, i, rw])
            pi_m = plsc.load_gather(gbuf, [bv, 2 + ccx, i, rw])
            tmask = r >= HEAD
            tw = jnp.maximum(r - HEAD, 0) * 4
            vi_t = plsc.load_gather(
                qtail_v, [lax.shift_right_logical(tw + cc, 4),
                          lax.bitwise_and(tw + cc, 15)])
            pi_t = plsc.load_gather(
                qtail_v, [lax.shift_right_logical(tw + ccx, 4),
                          lax.bitwise_and(tw + ccx, 15)])
            vi = jnp.where(tmask, vi_t, vi_m)
            pi = jnp.where(tmask, pi_t, pi_m)
            wp = rp * 4 + cc
            vp = plsc.load_gather(
                qp_v, [lax.shift_right_logical(wp, 4),
                       lax.bitwise_and(wp, 15)])
            wpx = lax.bitwise_xor(wp, 1)
            pp = plsc.load_gather(
                qp_v, [lax.shift_right_logical(wpx, 4),
                       lax.bitwise_and(wpx, 15)])
            q = vi + vp
            qq = pi + pp
            m = jnp.maximum(q, qq)
            t = jnp.exp(-jnp.abs(q - qq))
            q_v[pl.ds(pl.multiple_of(j * 512 + c * 16, 16), 16)] = (
                q - m - _log1p01(t))
            return ai + vi * vi, ap + vp * vp
        return q_body

    zero = jnp.zeros((16,), jnp.float32)
    acc_i, acc_p = zero, zero
    for j in range(NCHUNK):
        for cp in inflight[j]:
            cp.wait()
        lax.fori_loop(0, 16, pred_body_for(j, j % NBUF), 0)
        acc_i, acc_p = lax.fori_loop(
            0, 32, q_body_for(j, j % NBUF), (acc_i, acc_p))
        if j + NBUF < NCHUNK:
            inflight.append(issue(j + NBUF))

    cp_p = pltpu.async_copy(preds_v, preds_out.at[pl.ds(wid * 1024, 1024)],
                            sem_out)
    cp_q = pltpu.async_copy(q_v, q_out.at[pl.ds(wid * 2048, 2048)], sem_out)

    ri = jnp.sum(acc_i) * jnp.float32(BETA / 2.0)
    rp_ = jnp.sum(acc_p) * jnp.float32(ALPHA / 2.0)
    out16_v[pl.ds(0, 16)] = jnp.where(
        lanes == 0, ri, jnp.where(lanes == 1, rp_, jnp.float32(0.0)))
    pltpu.sync_copy(out16_v, reg_out.at[pl.ds(wid * 16, 16)])

    cp_p.wait()
    cp_q.wait()


_sc_call = functools.partial(
    pl.kernel,
    out_type=[
        jax.ShapeDtypeStruct((BATCH * 2,), jnp.float32),
        jax.ShapeDtypeStruct((BATCH * 4,), jnp.float32),
        jax.ShapeDtypeStruct((NW * 16,), jnp.float32),
    ],
    compiler_params=pltpu.CompilerParams(
        needs_layout_passes=False, use_tc_tiling_on_sc=False),
    mesh=plsc.VectorSubcoreMesh(
        core_axis_name="c", subcore_axis_name="s",
        num_cores=NUM_CORES, num_subcores=NUM_SUBCORES),
    scratch_types=[
        pltpu.VMEM((NCHUNK, 128), jnp.int32),
        pltpu.VMEM((NCHUNK, 128), jnp.int32),
        pltpu.VMEM((NCHUNK, 6, 128), jnp.int32),
        pltpu.VMEM((NBUF, 6, 128, 16), jnp.float32),
        pltpu.VMEM((NTAIL * 2 // 16, 16), jnp.float32),
        pltpu.VMEM((NTAIL * 4 // 16, 16), jnp.float32),
        pltpu.VMEM((250, 16), jnp.float32),
        pltpu.VMEM((BPW * 2,), jnp.float32),
        pltpu.VMEM((BPW * 4,), jnp.float32),
        pltpu.VMEM((16,), jnp.float32),
        pltpu.SemaphoreType.DMA,
        pltpu.SemaphoreType.DMA((NBUF,)),
        pltpu.SemaphoreType.DMA,
    ],
)(_sc_body)


def kernel(instances, predictors, labels, instance_emb, q_i_table, q_p_table):
    del labels
    we = (instance_emb[:HEAD].reshape(HEAD // 128, 128, 2)
          .transpose(0, 2, 1).reshape(-1, 16))
    wq = (q_i_table[:HEAD].reshape(HEAD // 128, 128, 4)
          .transpose(0, 2, 1).reshape(-1, 16))
    etail = instance_emb[HEAD:].reshape(-1, 16)
    qtail = q_i_table[HEAD:].reshape(-1, 16)
    qp16 = q_p_table.reshape(-1, 16)
    preds_flat, q_flat, regs = _sc_call(
        instances, predictors, we, wq, etail, qtail, qp16)
    predictions = preds_flat.reshape(BATCH, 2)
    q_params = q_flat.reshape(BATCH, 2, 2)
    parts = regs.reshape(NW, 16)
    reg_i = parts[:, 0].sum()
    reg_p = parts[:, 1].sum()
    return predictions, q_params, reg_i, reg_p


# speedup vs baseline: 19.8758x; 1.6071x over previous
import functools

import jax
import jax.numpy as jnp
from jax import lax
from jax.experimental import pallas as pl
from jax.experimental.pallas import tpu as pltpu
from jax.experimental.pallas import tpu_sc as plsc

NUM_CORES = 2
NUM_SUBCORES = 16
NW = NUM_CORES * NUM_SUBCORES
BATCH = 16384
BPW = BATCH // NW
NCHUNK = BPW // 128
NBUF = 3

NROWS = 1000000
HEAD = (NROWS // 128) * 128
NTAIL = NROWS - HEAD

ALPHA = 4.0
BETA = 400.0

_LOG1P_C = (
    2.2159764915272586e-07,
    0.9999702432977375,
    -0.49933394898195,
    0.3275117137018214,
    -0.22396689942949047,
    0.13198966239919324,
    -0.05326747773338258,
    0.010243828631143087,
)


def _log1p01(t):
    acc = jnp.full((16,), _LOG1P_C[-1], jnp.float32)
    for c in _LOG1P_C[-2::-1]:
        acc = acc * t + jnp.float32(c)
    return acc


def _sc_body(inst_hbm, pred_hbm, we_hbm, wq_hbm, etail_hbm, qtail_hbm, qp_hbm,
             preds_out, q_out, reg_out,
             idx_i, idx_p, blk, gbuf, etail_v, qtail_v, qp_v,
             preds_v, q_v, out16_v,
             sem_in, sem_g, sem_out):
    cid = lax.axis_index("c")
    sid = lax.axis_index("s")
    wid = cid * NUM_SUBCORES + sid
    base = wid * BPW
    lanes = lax.iota(jnp.int32, 16)

    cps = [pltpu.async_copy(etail_hbm, etail_v, sem_in),
           pltpu.async_copy(qtail_hbm, qtail_v, sem_in),
           pltpu.async_copy(qp_hbm, qp_v, sem_in)]
    for j in range(NCHUNK):
        cps.append(pltpu.async_copy(
            inst_hbm.at[pl.ds(base + 128 * j, 128)], idx_i.at[j], sem_in))
        cps.append(pltpu.async_copy(
            pred_hbm.at[pl.ds(base + 128 * j, 128)], idx_p.at[j], sem_in))
    for cp in cps:
        cp.wait()

    def blk_body(k, carry):
        row = jnp.full((16,), lax.shift_right_logical(k, 3), jnp.int32)
        col = lax.bitwise_and(k, 7) * 16 + lanes
        v = plsc.load_gather(idx_i, [row, col])
        t = lax.shift_right_logical(v, 7)
        lo = lax.bitwise_and(lax.shift_right_logical(v, 4), 7)
        inhead = v < HEAD
        e_base = t * 16 + lo
        q_base = t * 32 + lo
        zero = jnp.zeros((16,), jnp.int32)
        for m in range(2):
            plsc.store_scatter(
                blk, [row, jnp.full((16,), m, jnp.int32), col],
                jnp.where(inhead, e_base + 8 * m, zero))
        for m in range(4):
            plsc.store_scatter(
                blk, [row, jnp.full((16,), 2 + m, jnp.int32), col],
                jnp.where(inhead, q_base + 8 * m, zero))
        return carry

    lax.fori_loop(0, NCHUNK * 8, blk_body, 0)

    def issue(j):
        b = j % NBUF
        out = []
        for m in range(2):
            out.append(pltpu.async_copy(
                we_hbm.at[blk.at[j, m]], gbuf.at[b, m], sem_g.at[b]))
        for m in range(4):
            out.append(pltpu.async_copy(
                wq_hbm.at[blk.at[j, 2 + m]], gbuf.at[b, 2 + m], sem_g.at[b]))
        return out

    inflight = [issue(j) for j in range(NBUF)]

    def pred_body_for(j, b):
        def pred_body(c, carry):
            flat = c * 16 + lanes
            i = lax.bitwise_and(flat, 127)
            cc = lax.shift_right_logical(flat, 7)
            jv = jnp.full((16,), j, jnp.int32)
            bv = jnp.full((16,), b, jnp.int32)
            r = plsc.load_gather(idx_i, [jv, i])
            x_main = plsc.load_gather(
                gbuf, [bv, cc, i, lax.bitwise_and(r, 15)])
            tmask = r >= HEAD
            tw = jnp.maximum(r - HEAD, 0) * 2 + cc
            x_tail = plsc.load_gather(
                etail_v, [lax.shift_right_logical(tw, 4),
                          lax.bitwise_and(tw, 15)])
            x = jnp.where(tmask, x_tail, x_main)
            t = jnp.exp(-jnp.abs(x))
            preds_v[pl.ds(pl.multiple_of(j * 256 + c * 16, 16), 16)] = (
                jnp.minimum(x, 0.0) - _log1p01(t))
            return carry
        return pred_body

    def q_body_for(j, b):
        def q_body(c, carry):
            ai, ap = carry
            flat = c * 16 + lanes
            i = lax.bitwise_and(flat, 127)
            cc = lax.shift_right_logical(flat, 7)
            ccx = lax.bitwise_xor(cc, 1)
            jv = jnp.full((16,), j, jnp.int32)
            bv = jnp.full((16,), b, jnp.int32)
            r = plsc.load_gather(idx_i, [jv, i])
            rp = plsc.load_gather(idx_p, [jv, i])
            rw = lax.bitwise_and(r, 15)
            vi_m = plsc.load_gather(gbuf, [bv, 2 + cc, i, rw])
            pi_m = plsc.load_gather(gbuf, [bv, 2 + ccx, i, rw])
            tmask = r >= HEAD
            tw = jnp.maximum(r - HEAD, 0) * 4
            vi_t = plsc.load_gather(
                qtail_v, [lax.shift_right_logical(tw + cc, 4),
                          lax.bitwise_and(tw + cc, 15)])
            pi_t = plsc.load_gather(
                qtail_v, [lax.shift_right_logical(tw + ccx, 4),
                          lax.bitwise_and(tw + ccx, 15)])
            vi = jnp.where(tmask, vi_t, vi_m)
            pi = jnp.where(tmask, pi_t, pi_m)
            wp = rp * 4 + cc
            vp = plsc.load_gather(
                qp_v, [lax.shift_right_logical(wp, 4),
                       lax.bitwise_and(wp, 15)])
            wpx = lax.bitwise_xor(wp, 1)
            pp = plsc.load_gather(
                qp_v, [lax.shift_right_logical(wpx, 4),
                       lax.bitwise_and(wpx, 15)])
            q = vi + vp
            qq = pi + pp
            m = jnp.maximum(q, qq)
            t = jnp.exp(-jnp.abs(q - qq))
            off = (j * 256 + c * 16
                   + lax.shift_right_logical(c, 4) * (1024 - 256))
            q_v[pl.ds(pl.multiple_of(off, 16), 16)] = q - m - _log1p01(t)
            return ai + vi * vi, ap + vp * vp
        return q_body

    zero = jnp.zeros((16,), jnp.float32)
    acc_i, acc_p = zero, zero
    for j in range(NCHUNK):
        for cp in inflight[j]:
            cp.wait()
        lax.fori_loop(0, 16, pred_body_for(j, j % NBUF), 0)
        acc_i, acc_p = lax.fori_loop(
            0, 32, q_body_for(j, j % NBUF), (acc_i, acc_p))
        if j + NBUF < NCHUNK:
            inflight.append(issue(j + NBUF))

    cp_p = pltpu.async_copy(preds_v, preds_out.at[pl.ds(wid * 1024, 1024)],
                            sem_out)
    cp_q0 = pltpu.async_copy(q_v.at[pl.ds(0, 1024)],
                             q_out.at[pl.ds(wid * 1024, 1024)], sem_out)
    cp_q1 = pltpu.async_copy(q_v.at[pl.ds(1024, 1024)],
                             q_out.at[pl.ds(BATCH * 2 + wid * 1024, 1024)],
                             sem_out)

    ri = jnp.sum(acc_i) * jnp.float32(BETA / 2.0)
    rp_ = jnp.sum(acc_p) * jnp.float32(ALPHA / 2.0)
    out16_v[pl.ds(0, 16)] = jnp.where(
        lanes == 0, ri, jnp.where(lanes == 1, rp_, jnp.float32(0.0)))
    pltpu.sync_copy(out16_v, reg_out.at[pl.ds(wid * 16, 16)])

    cp_p.wait()
    cp_q0.wait()
    cp_q1.wait()


_sc_call = functools.partial(
    pl.kernel,
    out_type=[
        jax.ShapeDtypeStruct((BATCH * 2,), jnp.float32),
        jax.ShapeDtypeStruct((BATCH * 4,), jnp.float32),
        jax.ShapeDtypeStruct((NW * 16,), jnp.float32),
    ],
    compiler_params=pltpu.CompilerParams(
        needs_layout_passes=False, use_tc_tiling_on_sc=False),
    mesh=plsc.VectorSubcoreMesh(
        core_axis_name="c", subcore_axis_name="s",
        num_cores=NUM_CORES, num_subcores=NUM_SUBCORES),
    scratch_types=[
        pltpu.VMEM((NCHUNK, 128), jnp.int32),
        pltpu.VMEM((NCHUNK, 128), jnp.int32),
        pltpu.VMEM((NCHUNK, 6, 128), jnp.int32),
        pltpu.VMEM((NBUF, 6, 128, 16), jnp.float32),
        pltpu.VMEM((NTAIL * 2 // 16, 16), jnp.float32),
        pltpu.VMEM((NTAIL * 4 // 16, 16), jnp.float32),
        pltpu.VMEM((250, 16), jnp.float32),
        pltpu.VMEM((BPW * 2,), jnp.float32),
        pltpu.VMEM((BPW * 4,), jnp.float32),
        pltpu.VMEM((16,), jnp.float32),
        pltpu.SemaphoreType.DMA,
        pltpu.SemaphoreType.DMA((NBUF,)),
        pltpu.SemaphoreType.DMA,
    ],
)(_sc_body)


def kernel(instances, predictors, labels, instance_emb, q_i_table, q_p_table):
    del labels
    we = (instance_emb[:HEAD].reshape(HEAD // 128, 128, 2)
          .transpose(0, 2, 1).reshape(-1, 16))
    wq = (q_i_table[:HEAD].reshape(HEAD // 128, 128, 4)
          .transpose(0, 2, 1).reshape(-1, 16))
    etail = instance_emb[HEAD:].reshape(-1, 16)
    qtail = q_i_table[HEAD:].reshape(-1, 16)
    qp16 = q_p_table.reshape(-1, 16)
    preds_flat, q_flat, regs = _sc_call(
        instances, predictors, we, wq, etail, qtail, qp16)
    predictions = (preds_flat.reshape(BATCH // 128, 2, 128)
                   .transpose(0, 2, 1).reshape(BATCH, 2))
    q_params = (q_flat.reshape(2, BATCH // 128, 2, 128)
                .transpose(1, 3, 0, 2).reshape(BATCH, 2, 2))
    parts = regs.reshape(NW, 16)
    reg_i = parts[:, 0].sum()
    reg_p = parts[:, 1].sum()
    return predictions, q_params, reg_i, reg_p


# speedup vs baseline: 29.8316x; 1.5009x over previous
import functools

import jax
import jax.numpy as jnp
from jax import lax
from jax.experimental import pallas as pl
from jax.experimental.pallas import tpu as pltpu
from jax.experimental.pallas import tpu_sc as plsc

NUM_CORES = 2
NUM_SUBCORES = 16
NW = NUM_CORES * NUM_SUBCORES
BATCH = 16384
BPW = BATCH // NW
NCHUNK = BPW // 128
NBUF = 3

NROWS = 1000000
HEAD = (NROWS // 128) * 128
NTAIL = NROWS - HEAD

ALPHA = 4.0
BETA = 400.0

_LOG1P_C = (
    2.2159764915272586e-07,
    0.9999702432977375,
    -0.49933394898195,
    0.3275117137018214,
    -0.22396689942949047,
    0.13198966239919324,
    -0.05326747773338258,
    0.010243828631143087,
)


def _log1p01(t):
    acc = jnp.full((16,), _LOG1P_C[-1], jnp.float32)
    for c in _LOG1P_C[-2::-1]:
        acc = acc * t + jnp.float32(c)
    return acc


def _sc_body(inst_hbm, pred_hbm, we_hbm, wq_hbm, qp_hbm,
             preds_out, q_out, reg_out,
             idx_i, idx_p, blk, gbuf, qp_v,
             preds_v, q_v, out16_v,
             sem_in, sem_g, sem_out):
    cid = lax.axis_index("c")
    sid = lax.axis_index("s")
    wid = cid * NUM_SUBCORES + sid
    base = wid * BPW
    lanes = lax.iota(jnp.int32, 16)

    cps = [pltpu.async_copy(qp_hbm, qp_v, sem_in)]
    for j in range(NCHUNK):
        cps.append(pltpu.async_copy(
            inst_hbm.at[pl.ds(base + 128 * j, 128)], idx_i.at[j], sem_in))
        cps.append(pltpu.async_copy(
            pred_hbm.at[pl.ds(base + 128 * j, 128)], idx_p.at[j], sem_in))
    for cp in cps:
        cp.wait()

    def blk_body(k, carry):
        row = jnp.full((16,), lax.shift_right_logical(k, 3), jnp.int32)
        col = lax.bitwise_and(k, 7) * 16 + lanes
        v = plsc.load_gather(idx_i, [row, col])
        g = lax.shift_right_logical(v, 4)
        for m in range(2):
            plsc.store_scatter(
                blk, [row, jnp.full((16,), m, jnp.int32), col],
                g + (NROWS // 16) * m)
        for m in range(4):
            plsc.store_scatter(
                blk, [row, jnp.full((16,), 2 + m, jnp.int32), col],
                g + (NROWS // 16) * m)
        return carry

    lax.fori_loop(0, NCHUNK * 8, blk_body, 0)

    def issue(j):
        b = j % NBUF
        out = []
        for m in range(2):
            out.append(pltpu.async_copy(
                we_hbm.at[blk.at[j, m]], gbuf.at[b, m], sem_g.at[b]))
        for m in range(4):
            out.append(pltpu.async_copy(
                wq_hbm.at[blk.at[j, 2 + m]], gbuf.at[b, 2 + m], sem_g.at[b]))
        return out

    inflight = [issue(j) for j in range(NBUF)]

    def pred_body_for(j, b):
        def pred_body(c, carry):
            flat = c * 16 + lanes
            i = lax.bitwise_and(flat, 127)
            cc = lax.shift_right_logical(flat, 7)
            jv = jnp.full((16,), j, jnp.int32)
            bv = jnp.full((16,), b, jnp.int32)
            r = plsc.load_gather(idx_i, [jv, i])
            x = plsc.load_gather(gbuf, [bv, cc, i, lax.bitwise_and(r, 15)])
            t = jnp.exp(-jnp.abs(x))
            preds_v[pl.ds(pl.multiple_of(j * 256 + c * 16, 16), 16)] = (
                jnp.minimum(x, 0.0) - _log1p01(t))
            return carry
        return pred_body

    def q_body_for(j, b):
        def q_body(c, carry):
            ai, ap = carry
            flat = c * 16 + lanes
            i = lax.bitwise_and(flat, 127)
            cc = lax.shift_right_logical(flat, 7)
            ccx = lax.bitwise_xor(cc, 1)
            jv = jnp.full((16,), j, jnp.int32)
            bv = jnp.full((16,), b, jnp.int32)
            r = plsc.load_gather(idx_i, [jv, i])
            rp = plsc.load_gather(idx_p, [jv, i])
            rw = lax.bitwise_and(r, 15)
            vi = plsc.load_gather(gbuf, [bv, 2 + cc, i, rw])
            pi = plsc.load_gather(gbuf, [bv, 2 + ccx, i, rw])
            wp = rp * 4 + cc
            vp = plsc.load_gather(
                qp_v, [lax.shift_right_logical(wp, 4),
                       lax.bitwise_and(wp, 15)])
            wpx = lax.bitwise_xor(wp, 1)
            pp = plsc.load_gather(
                qp_v, [lax.shift_right_logical(wpx, 4),
                       lax.bitwise_and(wpx, 15)])
            q = vi + vp
            qq = pi + pp
            m = jnp.maximum(q, qq)
            t = jnp.exp(-jnp.abs(q - qq))
            off = (j * 256 + c * 16
                   + lax.shift_right_logical(c, 4) * (1024 - 256))
            q_v[pl.ds(pl.multiple_of(off, 16), 16)] = q - m - _log1p01(t)
            return ai + vi * vi, ap + vp * vp
        return q_body

    zero = jnp.zeros((16,), jnp.float32)
    acc_i, acc_p = zero, zero
    for j in range(NCHUNK):
        for cp in inflight[j]:
            cp.wait()
        lax.fori_loop(0, 16, pred_body_for(j, j % NBUF), 0)
        acc_i, acc_p = lax.fori_loop(
            0, 32, q_body_for(j, j % NBUF), (acc_i, acc_p))
        if j + NBUF < NCHUNK:
            inflight.append(issue(j + NBUF))

    cp_p = pltpu.async_copy(preds_v, preds_out.at[pl.ds(wid * 1024, 1024)],
                            sem_out)
    cp_q0 = pltpu.async_copy(q_v.at[pl.ds(0, 1024)],
                             q_out.at[pl.ds(wid * 1024, 1024)], sem_out)
    cp_q1 = pltpu.async_copy(q_v.at[pl.ds(1024, 1024)],
                             q_out.at[pl.ds(BATCH * 2 + wid * 1024, 1024)],
                             sem_out)

    ri = jnp.sum(acc_i) * jnp.float32(BETA / 2.0)
    rp_ = jnp.sum(acc_p) * jnp.float32(ALPHA / 2.0)
    out16_v[pl.ds(0, 16)] = jnp.where(
        lanes == 0, ri, jnp.where(lanes == 1, rp_, jnp.float32(0.0)))
    pltpu.sync_copy(out16_v, reg_out.at[pl.ds(wid * 16, 16)])

    cp_p.wait()
    cp_q0.wait()
    cp_q1.wait()


_sc_call = functools.partial(
    pl.kernel,
    out_type=[
        jax.ShapeDtypeStruct((BATCH * 2,), jnp.float32),
        jax.ShapeDtypeStruct((BATCH * 4,), jnp.float32),
        jax.ShapeDtypeStruct((NW * 16,), jnp.float32),
    ],
    compiler_params=pltpu.CompilerParams(
        needs_layout_passes=False, use_tc_tiling_on_sc=False),
    mesh=plsc.VectorSubcoreMesh(
        core_axis_name="c", subcore_axis_name="s",
        num_cores=NUM_CORES, num_subcores=NUM_SUBCORES),
    scratch_types=[
        pltpu.VMEM((NCHUNK, 128), jnp.int32),
        pltpu.VMEM((NCHUNK, 128), jnp.int32),
        pltpu.VMEM((NCHUNK, 6, 128), jnp.int32),
        pltpu.VMEM((NBUF, 6, 128, 16), jnp.float32),
        pltpu.VMEM((250, 16), jnp.float32),
        pltpu.VMEM((BPW * 2,), jnp.float32),
        pltpu.VMEM((BPW * 4,), jnp.float32),
        pltpu.VMEM((16,), jnp.float32),
        pltpu.SemaphoreType.DMA,
        pltpu.SemaphoreType.DMA((NBUF,)),
        pltpu.SemaphoreType.DMA,
    ],
)(_sc_body)


def kernel(instances, predictors, labels, instance_emb, q_i_table, q_p_table):
    del labels
    we = instance_emb.T.reshape(-1, 16)
    wq = q_i_table.T.reshape(-1, 16)
    qp16 = q_p_table.reshape(-1, 16)
    preds_flat, q_flat, regs = _sc_call(
        instances, predictors, we, wq, qp16)
    predictions = (preds_flat.reshape(BATCH // 128, 2, 128)
                   .transpose(0, 2, 1).reshape(BATCH, 2))
    q_params = (q_flat.reshape(2, BATCH // 128, 2, 128)
                .transpose(1, 3, 0, 2).reshape(BATCH, 2, 2))
    parts = regs.reshape(NW, 16)
    reg_i = parts[:, 0].sum()
    reg_p = parts[:, 1].sum()
    return predictions, q_params, reg_i, reg_p
